# speedup vs baseline: 18.8630x; 18.8630x over previous
import functools

import jax
import jax.numpy as jnp
from jax import lax
from jax.experimental import pallas as pl
from jax.experimental.pallas import tpu as pltpu
from jax.experimental.pallas import tpu_sc as plsc

N = 10000
E = 320000
D = 128

NC = 2
NS = 16
NW = NC * NS
CHUNK = 128

N_PAD = 10240
ROWS_PER_SUB = N_PAD // NS

NCH = -(-E // (NW * CHUNK))
E_PAD = NW * NCH * CHUNK

DEG_NCH = -(-(2 * E) // (NW * CHUNK))
DEG_E_PAD = NW * DEG_NCH * CHUNK
DEG_SH = 2 * N_PAD
DEG_PER_SUB = DEG_SH // NS

BR = 256
GRID = N_PAD // BR

_MESH = plsc.VectorSubcoreMesh(
    core_axis_name="c", subcore_axis_name="s", num_cores=NC, num_subcores=NS
)


def _wid():
    return lax.axis_index("s") * NC + lax.axis_index("c")


@functools.partial(
    pl.kernel,
    out_type=jax.ShapeDtypeStruct((NC, DEG_SH), jnp.float32),
    mesh=_MESH,
    scratch_types=[
        pltpu.VMEM((DEG_NCH, CHUNK), jnp.int32),
        pltpu.VMEM((CHUNK,), jnp.float32),
        pltpu.VMEM_SHARED((DEG_SH,), jnp.float32),
    ],
)
def _sc_degrees(idx_hbm, zeros_hbm, out_hbm, idx_v, ones_v, deg_sh):
    cid = lax.axis_index("c")
    sid = lax.axis_index("s")
    wid = _wid()

    pltpu.sync_copy(idx_hbm.at[wid], idx_v)
    for j in range(CHUNK // 16):
        ones_v[pl.ds(16 * j, 16)] = jnp.ones((16,), jnp.float32)

    sl = pl.ds(sid * DEG_PER_SUB, DEG_PER_SUB)
    pltpu.sync_copy(zeros_hbm.at[sl], deg_sh.at[sl])
    plsc.subcore_barrier()

    def body(j, carry):
        pltpu.sync_copy(ones_v, deg_sh.at[idx_v.at[j]], add=True)
        return carry

    lax.fori_loop(0, DEG_NCH, body, 0)
    plsc.subcore_barrier()

    pltpu.sync_copy(deg_sh.at[sl], out_hbm.at[cid, sl])


@functools.partial(
    pl.kernel,
    out_type=jax.ShapeDtypeStruct((NC, N_PAD, D), jnp.float32),
    mesh=_MESH,
    scratch_types=[
        pltpu.VMEM((NCH, CHUNK), jnp.int32),
        pltpu.VMEM((NCH, CHUNK), jnp.int32),
        pltpu.VMEM((CHUNK, D), jnp.float32),
        pltpu.VMEM_SHARED((N_PAD, D), jnp.float32),
        pltpu.SemaphoreType.DMA,
    ],
)
def _sc_msg(xs_hbm, row_hbm, col_hbm, zeros_hbm, out_hbm, row_v, col_v, buf, acc_sh, sem):
    cid = lax.axis_index("c")
    sid = lax.axis_index("s")
    wid = _wid()

    pltpu.sync_copy(row_hbm.at[wid], row_v)
    pltpu.sync_copy(col_hbm.at[wid], col_v)

    sl = pl.ds(sid * ROWS_PER_SUB, ROWS_PER_SUB)
    pltpu.sync_copy(zeros_hbm.at[sl], acc_sh.at[sl])
    plsc.subcore_barrier()

    def body(j, carry):
        pltpu.async_copy(xs_hbm.at[col_v.at[j]], buf, sem).wait()
        pltpu.sync_copy(buf, acc_sh.at[row_v.at[j]], add=True)
        return carry

    lax.fori_loop(0, NCH, body, 0)
    plsc.subcore_barrier()

    pltpu.sync_copy(acc_sh.at[sl], out_hbm.at[cid, sl])


def _col(v):
    return lax.transpose(v.reshape(1, BR), (1, 0))


def _tc_pre_body(h_ref, w_ref, dpo_ref, dpi_ref, xs_ref, db_ref, dinvb_ref):
    deg_in = dpi_ref[0, :] + dpi_ref[1, :] + 1.0
    d = lax.rsqrt(deg_in)
    deg_out = dpo_ref[0, :] + dpo_ref[1, :] + 1.0
    dinv = 1.0 / deg_out
    db = jnp.broadcast_to(_col(d), (BR, D))
    dinvb = jnp.broadcast_to(_col(dinv), (BR, D))
    x = lax.dot_general(
        h_ref[...], w_ref[...], (((1,), (0,)), ((), ())),
        preferred_element_type=jnp.float32,
    )
    xs_ref[...] = x * db
    db_ref[...] = db
    dinvb_ref[...] = dinvb


def _tc_pre(h, W, degp):
    return pl.pallas_call(
        _tc_pre_body,
        grid=(GRID,),
        in_specs=[
            pl.BlockSpec((BR, D), lambda i: (i, 0)),
            pl.BlockSpec((D, D), lambda i: (0, 0)),
            pl.BlockSpec((NC, BR), lambda i: (0, i)),
            pl.BlockSpec((NC, BR), lambda i: (0, i + GRID)),
        ],
        out_specs=[
            pl.BlockSpec((BR, D), lambda i: (i, 0)),
            pl.BlockSpec((BR, D), lambda i: (i, 0)),
            pl.BlockSpec((BR, D), lambda i: (i, 0)),
        ],
        out_shape=[
            jax.ShapeDtypeStruct((N_PAD, D), jnp.float32),
            jax.ShapeDtypeStruct((N_PAD, D), jnp.float32),
            jax.ShapeDtypeStruct((N_PAD, D), jnp.float32),
        ],
    )(h, W, degp, degp)


def _step_update(accp_ref, xs_ref, h_ref, db_ref, dinvb_ref, b_ref, root_ref):
    acc = accp_ref[0] + accp_ref[1] + xs_ref[...]
    h_msg = db_ref[...] * acc + b_ref[...]
    root_c = jax.nn.relu(h_ref[...] + root_ref[...]) * dinvb_ref[...]
    return h_msg + root_c


def _tc_step_body(accp_ref, xs_ref, h_ref, db_ref, dinvb_ref, b_ref, root_ref,
                  w_ref, hn_ref, xsn_ref):
    h_new = _step_update(accp_ref, xs_ref, h_ref, db_ref, dinvb_ref, b_ref, root_ref)
    hn_ref[...] = h_new
    x = lax.dot_general(
        h_new, w_ref[...], (((1,), (0,)), ((), ())),
        preferred_element_type=jnp.float32,
    )
    xsn_ref[...] = x * db_ref[...]


def _tc_step(accp, xs, h, db, dinvb, b2, root, W):
    return pl.pallas_call(
        _tc_step_body,
        grid=(GRID,),
        in_specs=[
            pl.BlockSpec((NC, BR, D), lambda i: (0, i, 0)),
            pl.BlockSpec((BR, D), lambda i: (i, 0)),
            pl.BlockSpec((BR, D), lambda i: (i, 0)),
            pl.BlockSpec((BR, D), lambda i: (i, 0)),
            pl.BlockSpec((BR, D), lambda i: (i, 0)),
            pl.BlockSpec((1, D), lambda i: (0, 0)),
            pl.BlockSpec((1, D), lambda i: (0, 0)),
            pl.BlockSpec((D, D), lambda i: (0, 0)),
        ],
        out_specs=[
            pl.BlockSpec((BR, D), lambda i: (i, 0)),
            pl.BlockSpec((BR, D), lambda i: (i, 0)),
        ],
        out_shape=[
            jax.ShapeDtypeStruct((N_PAD, D), jnp.float32),
            jax.ShapeDtypeStruct((N_PAD, D), jnp.float32),
        ],
    )(accp, xs, h, db, dinvb, b2, root, W)


def _tc_final_body(accp_ref, xs_ref, h_ref, db_ref, dinvb_ref, b_ref, root_ref,
                   hn_ref):
    hn_ref[...] = _step_update(accp_ref, xs_ref, h_ref, db_ref, dinvb_ref,
                               b_ref, root_ref)


def _tc_final(accp, xs, h, db, dinvb, b2, root):
    return pl.pallas_call(
        _tc_final_body,
        grid=(GRID,),
        in_specs=[
            pl.BlockSpec((NC, BR, D), lambda i: (0, i, 0)),
            pl.BlockSpec((BR, D), lambda i: (i, 0)),
            pl.BlockSpec((BR, D), lambda i: (i, 0)),
            pl.BlockSpec((BR, D), lambda i: (i, 0)),
            pl.BlockSpec((BR, D), lambda i: (i, 0)),
            pl.BlockSpec((1, D), lambda i: (0, 0)),
            pl.BlockSpec((1, D), lambda i: (0, 0)),
        ],
        out_specs=pl.BlockSpec((BR, D), lambda i: (i, 0)),
        out_shape=jax.ShapeDtypeStruct((N_PAD, D), jnp.float32),
    )(accp, xs, h, db, dinvb, b2, root)


def _pad_idx(idx, total):
    npad = total - idx.shape[0]
    sent = N + (jnp.arange(npad, dtype=jnp.int32) % (N_PAD - N))
    return jnp.concatenate([idx, sent])


@jax.jit
def kernel(in_feat, edge_index, W, b, root_emb):
    row = edge_index[0].astype(jnp.int32)
    col = edge_index[1].astype(jnp.int32)

    row3 = _pad_idx(row, E_PAD).reshape(NW, NCH, CHUNK)
    col3 = _pad_idx(col, E_PAD).reshape(NW, NCH, CHUNK)
    deg_idx = _pad_idx(
        jnp.concatenate([row, col + N_PAD]), DEG_E_PAD
    ).reshape(NW, DEG_NCH, CHUNK)

    h0 = jnp.zeros((N_PAD, D), jnp.float32).at[:N].set(in_feat)
    zeros_feat = jnp.zeros((N_PAD, D), jnp.float32)
    zeros_deg = jnp.zeros((DEG_SH,), jnp.float32)
    b2 = b.reshape(1, D)

    degp = _sc_degrees(deg_idx, zeros_deg)
    xs0, db, dinvb = _tc_pre(h0, W, degp)

    accp0 = _sc_msg(xs0, row3, col3, zeros_feat)
    h1, xs1 = _tc_step(accp0, xs0, h0, db, dinvb, b2, root_emb, W)

    accp1 = _sc_msg(xs1, row3, col3, zeros_feat)
    h2 = _tc_final(accp1, xs1, h1, db, dinvb, b2, root_emb)

    return h2[:N]


# speedup vs baseline: 23.4339x; 1.2423x over previous
import functools

import jax
import jax.numpy as jnp
from jax import lax
from jax.experimental import pallas as pl
from jax.experimental.pallas import tpu as pltpu
from jax.experimental.pallas import tpu_sc as plsc

N = 10000
E = 320000
D = 128

NC = 2
NS = 16
NW = NC * NS
CHUNK = 128

N_PAD = 10240
ROWS_PER_SUB = N_PAD // NS

NCH = -(-E // (NW * CHUNK))
E_PAD = NW * NCH * CHUNK

DEG_NCH = -(-(2 * E) // (NW * CHUNK))
DEG_E_PAD = NW * DEG_NCH * CHUNK
DEG_SH = 2 * N_PAD
DEG_PER_SUB = DEG_SH // NS

BR = 256
GRID = N_PAD // BR

_MESH = plsc.VectorSubcoreMesh(
    core_axis_name="c", subcore_axis_name="s", num_cores=NC, num_subcores=NS
)


def _wid():
    return lax.axis_index("s") * NC + lax.axis_index("c")


@functools.partial(
    pl.kernel,
    out_type=jax.ShapeDtypeStruct((NC, DEG_SH), jnp.float32),
    mesh=_MESH,
    scratch_types=[
        pltpu.VMEM((DEG_NCH, CHUNK), jnp.int32),
        pltpu.VMEM((CHUNK,), jnp.float32),
        pltpu.VMEM_SHARED((DEG_SH,), jnp.float32),
    ],
)
def _sc_degrees(idx_hbm, zeros_hbm, out_hbm, idx_v, ones_v, deg_sh):
    cid = lax.axis_index("c")
    sid = lax.axis_index("s")
    wid = _wid()

    pltpu.sync_copy(idx_hbm.at[wid], idx_v)
    for j in range(CHUNK // 16):
        ones_v[pl.ds(16 * j, 16)] = jnp.ones((16,), jnp.float32)

    sl = pl.ds(sid * DEG_PER_SUB, DEG_PER_SUB)
    pltpu.sync_copy(zeros_hbm.at[sl], deg_sh.at[sl])
    plsc.subcore_barrier()

    def body(j, carry):
        pltpu.sync_copy(ones_v, deg_sh.at[idx_v.at[j]], add=True)
        return carry

    lax.fori_loop(0, DEG_NCH, body, 0)
    plsc.subcore_barrier()

    pltpu.sync_copy(deg_sh.at[sl], out_hbm.at[cid, sl])


@functools.partial(
    pl.kernel,
    out_type=jax.ShapeDtypeStruct((NC, N_PAD, D), jnp.float32),
    mesh=_MESH,
    scratch_types=[
        pltpu.VMEM((NCH, CHUNK), jnp.int32),
        pltpu.VMEM((1, CHUNK), jnp.int32),
        pltpu.VMEM((1, CHUNK), jnp.int32),
        pltpu.VMEM((CHUNK, D), jnp.float32),
        pltpu.VMEM((CHUNK, D), jnp.float32),
        pltpu.VMEM_SHARED((N_PAD, D), jnp.float32),
        pltpu.SemaphoreType.DMA,
        pltpu.SemaphoreType.DMA,
    ],
)
def _sc_msg(xs_hbm, row_hbm, col_hbm, zeros_hbm, out_hbm, col_v, rb0, rb1,
            buf0, buf1, acc_sh, sem_g, sem_r):
    cid = lax.axis_index("c")
    sid = lax.axis_index("s")
    wid = _wid()

    pltpu.sync_copy(col_hbm.at[wid], col_v)

    def _gather(j, buf):
        pltpu.async_copy(xs_hbm.at[col_v.at[j]], buf, sem_g)

    def _wait_gather(buf):
        pltpu.make_async_copy(xs_hbm.at[pl.ds(0, CHUNK)], buf, sem_g).wait()

    def _fetch_row(j, rb):
        pltpu.async_copy(row_hbm.at[wid, pl.ds(j, 1)], rb, sem_r)

    def _wait_row(rb):
        pltpu.make_async_copy(row_hbm.at[wid, pl.ds(0, 1)], rb, sem_r).wait()

    def _scatter(buf, rb):
        pltpu.sync_copy(buf, acc_sh.at[rb.at[0]], add=True)

    _fetch_row(0, rb0)
    _fetch_row(1, rb1)
    _gather(0, buf0)

    sl = pl.ds(sid * ROWS_PER_SUB, ROWS_PER_SUB)
    pltpu.sync_copy(zeros_hbm.at[sl], acc_sh.at[sl])
    plsc.subcore_barrier()
    _wait_row(rb0)

    def body(i, carry):
        j = i * 2
        _wait_gather(buf0)
        _gather(j + 1, buf1)
        _scatter(buf0, rb0)
        _fetch_row(j + 2, rb0)
        _wait_row(rb1)
        _wait_gather(buf1)
        _gather(j + 2, buf0)
        _scatter(buf1, rb1)
        _fetch_row(j + 3, rb1)
        _wait_row(rb0)
        return carry

    lax.fori_loop(0, (NCH - 1) // 2, body, 0)
    _wait_gather(buf0)
    _scatter(buf0, rb0)
    _wait_row(rb1)
    plsc.subcore_barrier()

    pltpu.sync_copy(acc_sh.at[sl], out_hbm.at[cid, sl])


def _col(v):
    return lax.transpose(v.reshape(1, BR), (1, 0))


def _tc_pre_body(h_ref, w_ref, dpo_ref, dpi_ref, xs_ref, db_ref, dinvb_ref):
    deg_in = dpi_ref[0, :] + dpi_ref[1, :] + 1.0
    d = lax.rsqrt(deg_in)
    deg_out = dpo_ref[0, :] + dpo_ref[1, :] + 1.0
    dinv = 1.0 / deg_out
    db = jnp.broadcast_to(_col(d), (BR, D))
    dinvb = jnp.broadcast_to(_col(dinv), (BR, D))
    x = lax.dot_general(
        h_ref[...], w_ref[...], (((1,), (0,)), ((), ())),
        preferred_element_type=jnp.float32,
    )
    xs_ref[...] = x * db
    db_ref[...] = db
    dinvb_ref[...] = dinvb


def _tc_pre(h, W, degp):
    return pl.pallas_call(
        _tc_pre_body,
        grid=(GRID,),
        in_specs=[
            pl.BlockSpec((BR, D), lambda i: (i, 0)),
            pl.BlockSpec((D, D), lambda i: (0, 0)),
            pl.BlockSpec((NC, BR), lambda i: (0, i)),
            pl.BlockSpec((NC, BR), lambda i: (0, i + GRID)),
        ],
        out_specs=[
            pl.BlockSpec((BR, D), lambda i: (i, 0)),
            pl.BlockSpec((BR, D), lambda i: (i, 0)),
            pl.BlockSpec((BR, D), lambda i: (i, 0)),
        ],
        out_shape=[
            jax.ShapeDtypeStruct((N_PAD, D), jnp.float32),
            jax.ShapeDtypeStruct((N_PAD, D), jnp.float32),
            jax.ShapeDtypeStruct((N_PAD, D), jnp.float32),
        ],
    )(h, W, degp, degp)


def _step_update(accp_ref, xs_ref, h_ref, db_ref, dinvb_ref, b_ref, root_ref):
    acc = accp_ref[0] + accp_ref[1] + xs_ref[...]
    h_msg = db_ref[...] * acc + b_ref[...]
    root_c = jax.nn.relu(h_ref[...] + root_ref[...]) * dinvb_ref[...]
    return h_msg + root_c


def _tc_step_body(accp_ref, xs_ref, h_ref, db_ref, dinvb_ref, b_ref, root_ref,
                  w_ref, hn_ref, xsn_ref):
    h_new = _step_update(accp_ref, xs_ref, h_ref, db_ref, dinvb_ref, b_ref, root_ref)
    hn_ref[...] = h_new
    x = lax.dot_general(
        h_new, w_ref[...], (((1,), (0,)), ((), ())),
        preferred_element_type=jnp.float32,
    )
    xsn_ref[...] = x * db_ref[...]


def _tc_step(accp, xs, h, db, dinvb, b2, root, W):
    return pl.pallas_call(
        _tc_step_body,
        grid=(GRID,),
        in_specs=[
            pl.BlockSpec((NC, BR, D), lambda i: (0, i, 0)),
            pl.BlockSpec((BR, D), lambda i: (i, 0)),
            pl.BlockSpec((BR, D), lambda i: (i, 0)),
            pl.BlockSpec((BR, D), lambda i: (i, 0)),
            pl.BlockSpec((BR, D), lambda i: (i, 0)),
            pl.BlockSpec((1, D), lambda i: (0, 0)),
            pl.BlockSpec((1, D), lambda i: (0, 0)),
            pl.BlockSpec((D, D), lambda i: (0, 0)),
        ],
        out_specs=[
            pl.BlockSpec((BR, D), lambda i: (i, 0)),
            pl.BlockSpec((BR, D), lambda i: (i, 0)),
        ],
        out_shape=[
            jax.ShapeDtypeStruct((N_PAD, D), jnp.float32),
            jax.ShapeDtypeStruct((N_PAD, D), jnp.float32),
        ],
    )(accp, xs, h, db, dinvb, b2, root, W)


def _tc_final_body(accp_ref, xs_ref, h_ref, db_ref, dinvb_ref, b_ref, root_ref,
                   hn_ref):
    hn_ref[...] = _step_update(accp_ref, xs_ref, h_ref, db_ref, dinvb_ref,
                               b_ref, root_ref)


def _tc_final(accp, xs, h, db, dinvb, b2, root):
    return pl.pallas_call(
        _tc_final_body,
        grid=(GRID,),
        in_specs=[
            pl.BlockSpec((NC, BR, D), lambda i: (0, i, 0)),
            pl.BlockSpec((BR, D), lambda i: (i, 0)),
            pl.BlockSpec((BR, D), lambda i: (i, 0)),
            pl.BlockSpec((BR, D), lambda i: (i, 0)),
            pl.BlockSpec((BR, D), lambda i: (i, 0)),
            pl.BlockSpec((1, D), lambda i: (0, 0)),
            pl.BlockSpec((1, D), lambda i: (0, 0)),
        ],
        out_specs=pl.BlockSpec((BR, D), lambda i: (i, 0)),
        out_shape=jax.ShapeDtypeStruct((N_PAD, D), jnp.float32),
    )(accp, xs, h, db, dinvb, b2, root)


def _pad_idx(idx, total):
    npad = total - idx.shape[0]
    sent = N + (jnp.arange(npad, dtype=jnp.int32) % (N_PAD - N))
    return jnp.concatenate([idx, sent])


@jax.jit
def kernel(in_feat, edge_index, W, b, root_emb):
    row = edge_index[0].astype(jnp.int32)
    col = edge_index[1].astype(jnp.int32)

    row3 = _pad_idx(row, E_PAD).reshape(NW, NCH, CHUNK)
    row3 = jnp.concatenate(
        [row3, jnp.full((NW, 1, CHUNK), N, jnp.int32)], axis=1)
    col3 = _pad_idx(col, E_PAD).reshape(NW, NCH, CHUNK)
    deg_idx = _pad_idx(
        jnp.concatenate([row, col + N_PAD]), DEG_E_PAD
    ).reshape(NW, DEG_NCH, CHUNK)

    h0 = jnp.zeros((N_PAD, D), jnp.float32).at[:N].set(in_feat)
    zeros_feat = jnp.zeros((N_PAD, D), jnp.float32)
    zeros_deg = jnp.zeros((DEG_SH,), jnp.float32)
    b2 = b.reshape(1, D)

    degp = _sc_degrees(deg_idx, zeros_deg)
    xs0, db, dinvb = _tc_pre(h0, W, degp)

    accp0 = _sc_msg(xs0, row3, col3, zeros_feat)
    h1, xs1 = _tc_step(accp0, xs0, h0, db, dinvb, b2, root_emb, W)

    accp1 = _sc_msg(xs1, row3, col3, zeros_feat)
    h2 = _tc_final(accp1, xs1, h1, db, dinvb, b2, root_emb)

    return h2[:N]
